# Initial kernel scaffold; baseline (speedup 1.0000x reference)
import jax
import jax.numpy as jnp
from jax.experimental import pallas as pl


def kernel(features, adj, mean_fewshot_logits, W_enc, W_dec, toy_keys, toy_labels):
    raise NotImplementedError("write your pallas kernel here")


# speedup vs baseline: 2.9629x; 2.9629x over previous
import functools

import jax
import jax.numpy as jnp
from jax.experimental import pallas as pl

N, D, M, C, E, K = 4096, 256, 8192, 6, 256, 5
BA = 128
BC = 256
BP = 512
NEG = -1e30
BIGI = 2**30


def _bmm(a, b, dims):
    return jax.lax.dot_general(
        a.astype(jnp.bfloat16), b.astype(jnp.bfloat16), dims,
        preferred_element_type=jnp.float32)


def _fw_body(f_ref, we_ref, fw_ref):
    fw_ref[...] = _bmm(f_ref[...], we_ref[...], (((1,), (0,)), ((), ())))


def _c16_body(tk_ref, tl_ref, mfl_ref, wd_ref, c16_ref):
    tv = jax.lax.dot_general(
        tk_ref[...], wd_ref[...], (((1,), (0,)), ((), ())),
        precision=jax.lax.Precision.HIGHEST,
        preferred_element_type=jnp.float32) * 0.5
    lab = tl_ref[...]
    m = jnp.max(lab, axis=1, keepdims=True)
    ci = jax.lax.broadcasted_iota(jnp.int32, lab.shape, 1)
    ji = jnp.min(jnp.where(lab == m, ci, BIGI), axis=1, keepdims=True)
    oh = (ci == ji).astype(jnp.float32)
    lg = jax.lax.dot_general(
        oh, mfl_ref[...], (((1,), (0,)), ((), ())),
        precision=jax.lax.Precision.HIGHEST,
        preferred_element_type=jnp.float32) * 0.1
    z = jnp.zeros((lab.shape[0], 16 - 2 * C), jnp.float32)
    c16_ref[...] = jnp.concatenate([tv, lg, z], axis=1)


def _main_body(adj_ref, fw_ref, tk_ref, wd_ref, c16_ref,
               idx_ref, u0_ref, r16_ref):
    a = adj_ref[...]
    rs = jnp.sum(a, axis=1, keepdims=True) + 1e-8
    an = a / rs
    p = jnp.maximum(
        _bmm(an, fw_ref[...], (((1,), (0,)), ((), ()))), 0.0)
    u0_ref[...] = jax.lax.dot_general(
        p, wd_ref[...], (((1,), (0,)), ((), ())),
        precision=jax.lax.Precision.HIGHEST,
        preferred_element_type=jnp.float32)
    sim = _bmm(p, tk_ref[...], (((1,), (1,)), ((), ())))
    col = jax.lax.broadcasted_iota(jnp.int32, sim.shape, 1)
    cnt = jnp.zeros(sim.shape, jnp.float32)
    idxs = []
    s = sim
    for _ in range(K):
        m = jnp.max(s, axis=1, keepdims=True)
        ji = jnp.min(jnp.where(s == m, col, BIGI), axis=1, keepdims=True)
        idxs.append(ji)
        hit = col == ji
        s = jnp.where(hit, NEG, s)
        cnt = cnt + hit.astype(jnp.float32)
    idx_ref[...] = jnp.concatenate(idxs, axis=1)
    r16_ref[...] = jax.lax.dot_general(
        cnt, c16_ref[...], (((1,), (0,)), ((), ())),
        precision=jax.lax.Precision.HIGHEST,
        preferred_element_type=jnp.float32)


def _chain_body(adj_ref, u_ref, o_ref):
    a = adj_ref[...]
    rs = jnp.sum(a, axis=1, keepdims=True) + 1e-8
    o_ref[...] = jax.lax.dot_general(
        a, u_ref[...], (((1,), (0,)), ((), ())),
        precision=jax.lax.Precision.HIGHEST,
        preferred_element_type=jnp.float32) / rs


def _chain3_body(adj_ref, u_ref, r16_ref, h_ref):
    a = adj_ref[...]
    rs = jnp.sum(a, axis=1, keepdims=True) + 1e-8
    u3 = jax.lax.dot_general(
        a, u_ref[...], (((1,), (0,)), ((), ())),
        precision=jax.lax.Precision.HIGHEST,
        preferred_element_type=jnp.float32) / rs
    h_ref[...] = 0.5 * u3 + r16_ref[..., 0:C]


def _final_body(adj_ref, h_ref, r16_ref, o_ref):
    a = adj_ref[...]
    rs = jnp.sum(a, axis=1, keepdims=True) + 1e-8
    dec = jax.lax.dot_general(
        a, h_ref[...], (((1,), (0,)), ((), ())),
        precision=jax.lax.Precision.HIGHEST,
        preferred_element_type=jnp.float32) / rs
    o_ref[...] = 0.5 * dec + r16_ref[..., C:2 * C]


def _full(shape):
    return pl.BlockSpec(shape, lambda i: (0,) * len(shape))


def _rows(b, w):
    return pl.BlockSpec((b, w), lambda i: (i, 0))


def kernel(features, adj, mean_fewshot_logits, W_enc, W_dec, toy_keys,
           toy_labels):
    f32 = jnp.float32

    fw = pl.pallas_call(
        _fw_body,
        grid=(N // BP,),
        in_specs=[_rows(BP, D), _full((D, E))],
        out_specs=_rows(BP, E),
        out_shape=jax.ShapeDtypeStruct((N, E), f32),
    )(features, W_enc)

    c16 = pl.pallas_call(
        _c16_body,
        grid=(M // BP,),
        in_specs=[_rows(BP, E), _rows(BP, C), _full((C, C)), _full((E, C))],
        out_specs=_rows(BP, 16),
        out_shape=jax.ShapeDtypeStruct((M, 16), f32),
    )(toy_keys, toy_labels, mean_fewshot_logits, W_dec)

    idx, u0, r16 = pl.pallas_call(
        _main_body,
        grid=(N // BA,),
        in_specs=[_rows(BA, N), _full((N, E)), _full((M, E)), _full((E, C)),
                  _full((M, 16))],
        out_specs=[_rows(BA, K), _rows(BA, C), _rows(BA, 16)],
        out_shape=[jax.ShapeDtypeStruct((N, K), jnp.int32),
                   jax.ShapeDtypeStruct((N, C), f32),
                   jax.ShapeDtypeStruct((N, 16), f32)],
    )(adj, fw, toy_keys, W_dec, c16)
    del idx

    chain = pl.pallas_call(
        _chain_body,
        grid=(N // BC,),
        in_specs=[_rows(BC, N), _full((N, C))],
        out_specs=_rows(BC, C),
        out_shape=jax.ShapeDtypeStruct((N, C), f32),
    )
    u1 = chain(adj, u0)
    u2 = chain(adj, u1)

    h = pl.pallas_call(
        _chain3_body,
        grid=(N // BC,),
        in_specs=[_rows(BC, N), _full((N, C)), _rows(BC, 16)],
        out_specs=_rows(BC, C),
        out_shape=jax.ShapeDtypeStruct((N, C), f32),
    )(adj, u2, r16)

    out = pl.pallas_call(
        _final_body,
        grid=(N // BC,),
        in_specs=[_rows(BC, N), _full((N, C)), _rows(BC, 16)],
        out_specs=_rows(BC, C),
        out_shape=jax.ShapeDtypeStruct((N, C), f32),
    )(adj, h, r16)
    return out


# speedup vs baseline: 5.5532x; 1.8742x over previous
import jax
import jax.numpy as jnp
from jax.experimental import pallas as pl

N, D, M, C, E, K = 4096, 256, 8192, 6, 256, 5
BA = 128
BC = 256
BP = 512
NEG = -1e30
BIGI = 2**30
f32 = jnp.float32
bf16 = jnp.bfloat16


def _bmm(a, b, dims):
    return jax.lax.dot_general(
        a.astype(bf16), b.astype(bf16), dims,
        preferred_element_type=f32)


def _fw_body(f_ref, we_ref, fw_ref):
    fw_ref[...] = _bmm(f_ref[...], we_ref[...],
                       (((1,), (0,)), ((), ()))).astype(bf16)


def _c16_body(tk_ref, tl_ref, mfl_ref, wd_ref, c16_ref):
    tv = jax.lax.dot_general(
        tk_ref[...], wd_ref[...], (((1,), (0,)), ((), ())),
        precision=jax.lax.Precision.HIGHEST,
        preferred_element_type=f32) * 0.5
    lab = tl_ref[...]
    m = jnp.max(lab, axis=1, keepdims=True)
    ci = jax.lax.broadcasted_iota(jnp.int32, lab.shape, 1)
    ji = jnp.min(jnp.where(lab == m, ci, BIGI), axis=1, keepdims=True)
    oh = (ci == ji).astype(f32)
    lg = jax.lax.dot_general(
        oh, mfl_ref[...], (((1,), (0,)), ((), ())),
        precision=jax.lax.Precision.HIGHEST,
        preferred_element_type=f32) * 0.1
    z = jnp.zeros((lab.shape[0], 16 - 2 * C), f32)
    c16_ref[...] = jnp.concatenate([tv, lg, z], axis=1)


def _main_body(adj_ref, fw_ref, tk_ref, wd_ref, c16_ref,
               idx_ref, u0_ref, r16_ref, abf_ref):
    a = adj_ref[...]
    rs = jnp.sum(a, axis=1, keepdims=True) + 1e-8
    ab = (a / rs).astype(bf16)
    abf_ref[...] = ab
    p = jnp.maximum(
        jax.lax.dot_general(ab, fw_ref[...], (((1,), (0,)), ((), ())),
                            preferred_element_type=f32), 0.0)
    u0_ref[...] = jax.lax.dot_general(
        p, wd_ref[...], (((1,), (0,)), ((), ())),
        precision=jax.lax.Precision.HIGHEST,
        preferred_element_type=f32)
    s = jax.lax.dot_general(
        p.astype(bf16), tk_ref[...], (((1,), (1,)), ((), ())),
        preferred_element_type=f32)
    col = jax.lax.broadcasted_iota(jnp.int32, s.shape, 1)
    m = jnp.max(s, axis=1, keepdims=True)
    idxs = []
    for k in range(K):
        eq = s == m
        idxs.append(jnp.min(jnp.where(eq, col, BIGI), axis=1, keepdims=True))
        s = jnp.where(eq, NEG, s)
        if k < K - 1:
            m = jnp.max(s, axis=1, keepdims=True)
    idx_ref[...] = jnp.concatenate(idxs, axis=1)
    cnt = (s == NEG).astype(bf16)
    r16_ref[...] = jax.lax.dot_general(
        cnt, c16_ref[...], (((1,), (0,)), ((), ())),
        preferred_element_type=f32)


def _chain_body(abf_ref, u_ref, o_ref):
    o_ref[...] = jax.lax.dot_general(
        abf_ref[...], u_ref[...].astype(bf16), (((1,), (0,)), ((), ())),
        preferred_element_type=f32)


def _chain3_body(abf_ref, u_ref, r16_ref, h_ref):
    u3 = jax.lax.dot_general(
        abf_ref[...], u_ref[...].astype(bf16), (((1,), (0,)), ((), ())),
        preferred_element_type=f32)
    h_ref[...] = 0.5 * u3 + r16_ref[..., 0:C]


def _final_body(abf_ref, h_ref, r16_ref, o_ref):
    dec = jax.lax.dot_general(
        abf_ref[...], h_ref[...].astype(bf16), (((1,), (0,)), ((), ())),
        preferred_element_type=f32)
    o_ref[...] = 0.5 * dec + r16_ref[..., C:2 * C]


def _full(shape):
    return pl.BlockSpec(shape, lambda i: (0,) * len(shape))


def _rows(b, w):
    return pl.BlockSpec((b, w), lambda i: (i, 0))


def kernel(features, adj, mean_fewshot_logits, W_enc, W_dec, toy_keys,
           toy_labels):
    fw = pl.pallas_call(
        _fw_body,
        grid=(N // BP,),
        in_specs=[_rows(BP, D), _full((D, E))],
        out_specs=_rows(BP, E),
        out_shape=jax.ShapeDtypeStruct((N, E), bf16),
    )(features, W_enc)

    c16 = pl.pallas_call(
        _c16_body,
        grid=(M // BP,),
        in_specs=[_rows(BP, E), _rows(BP, C), _full((C, C)), _full((E, C))],
        out_specs=_rows(BP, 16),
        out_shape=jax.ShapeDtypeStruct((M, 16), f32),
    )(toy_keys, toy_labels, mean_fewshot_logits, W_dec)

    tkb = toy_keys.astype(bf16)
    c16b = c16.astype(bf16)

    idx, u0, r16, abf = pl.pallas_call(
        _main_body,
        grid=(N // BA,),
        in_specs=[_rows(BA, N), _full((N, E)), _full((M, E)), _full((E, C)),
                  _full((M, 16))],
        out_specs=[_rows(BA, K), _rows(BA, C), _rows(BA, 16), _rows(BA, N)],
        out_shape=[jax.ShapeDtypeStruct((N, K), jnp.int32),
                   jax.ShapeDtypeStruct((N, C), f32),
                   jax.ShapeDtypeStruct((N, 16), f32),
                   jax.ShapeDtypeStruct((N, N), bf16)],
    )(adj, fw, tkb, W_dec, c16b)
    del idx

    chain = pl.pallas_call(
        _chain_body,
        grid=(N // BC,),
        in_specs=[_rows(BC, N), _full((N, C))],
        out_specs=_rows(BC, C),
        out_shape=jax.ShapeDtypeStruct((N, C), f32),
    )
    u1 = chain(abf, u0)
    u2 = chain(abf, u1)

    h = pl.pallas_call(
        _chain3_body,
        grid=(N // BC,),
        in_specs=[_rows(BC, N), _full((N, C)), _rows(BC, 16)],
        out_specs=_rows(BC, C),
        out_shape=jax.ShapeDtypeStruct((N, C), f32),
    )(abf, u2, r16)

    out = pl.pallas_call(
        _final_body,
        grid=(N // BC,),
        in_specs=[_rows(BC, N), _full((N, C)), _rows(BC, 16)],
        out_specs=_rows(BC, C),
        out_shape=jax.ShapeDtypeStruct((N, C), f32),
    )(abf, h, r16)
    return out
